# Initial kernel scaffold; baseline (speedup 1.0000x reference)
import functools

import jax
import jax.numpy as jnp
import numpy as np
from jax.experimental import pallas as pl
from jax.experimental.pallas import tpu as pltpu

_B, _C, _FD, _H, _L = 8, 10000, 8, 128, 4
_MIN_LS, _MAX_LS = -7.0, 3.0


def _gelu(v):
    return jax.nn.gelu(v, approximate=False)


def _ln(v, g, b, eps=1e-5):
    mu = jnp.mean(v, axis=-1, keepdims=True)
    var = jnp.mean((v - mu) ** 2, axis=-1, keepdims=True)
    return (v - mu) * jax.lax.rsqrt(var + eps) * g + b


def _main_body(x_ref, W1_ref, b1_ref, W2_ref, b2_ref, Wg_ref, bg_ref, gg_ref,
               betag_ref, ghln_ref, bhln_ref, Wh1_ref, bh1_ref, Wh2_ref, bh2_ref,
               out_ref):
    xb = x_ref[0]
    h = _gelu(jnp.dot(xb, W1_ref[...], preferred_element_type=jnp.float32) + b1_ref[...])
    h = jnp.dot(h, W2_ref[...], preferred_element_type=jnp.float32) + b2_ref[...]

    t = jax.lax.broadcasted_iota(jnp.float32, (_C, 1), 0)
    third = jnp.float32(1.0 / 3.0)
    s6 = jnp.float32(1.0 / np.sqrt(6.0))
    cd = jnp.where((t == 0.0) | (t == _C - 1.0), jnp.float32(0.5), third)
    cl = jnp.where(t == 0.0, jnp.float32(0.0),
                   jnp.where((t == 1.0) | (t == _C - 1.0), s6, third))
    cr = jnp.where(t == _C - 1.0, jnp.float32(0.0),
                   jnp.where((t == 0.0) | (t == _C - 2.0), s6, third))

    zrow = jnp.zeros((1, _H), jnp.float32)
    for l in range(_L):
        hl = jnp.dot(h, Wg_ref[l], preferred_element_type=jnp.float32) + bg_ref[l]
        prev = jnp.concatenate([zrow, hl[:-1, :]], axis=0)
        nxt = jnp.concatenate([hl[1:, :], zrow], axis=0)
        agg = cd * hl + cl * prev + cr * nxt
        h = jax.nn.relu(_ln(agg + h, gg_ref[l], betag_ref[l]))

    z = _ln(h, ghln_ref[...], bhln_ref[...])
    z = _gelu(jnp.dot(z, Wh1_ref[...], preferred_element_type=jnp.float32) + bh1_ref[...])
    z2 = jnp.dot(z, Wh2_ref[...], preferred_element_type=jnp.float32) + bh2_ref[...]
    col = jax.lax.broadcasted_iota(jnp.int32, (_C, 2), 1)
    z2 = jnp.where(col == 1, jnp.clip(z2, _MIN_LS, _MAX_LS), z2)
    out_ref[0] = z2


def _full(shape):
    return pl.BlockSpec(shape, lambda b: (0,) * len(shape))


@jax.jit
def _run(xs, W1, b1, W2, b2, Wg, bg, gg, betag, ghln, bhln, Wh1, bh1, Wh2, bh2):
    return pl.pallas_call(
        _main_body,
        grid=(_B,),
        in_specs=[
            pl.BlockSpec((1, _C, _FD), lambda b: (b, 0, 0)),
            _full((_FD, _H)), _full((1, _H)),
            _full((_H, _H)), _full((1, _H)),
            _full((_L, _H, _H)), _full((_L, 1, _H)),
            _full((_L, 1, _H)), _full((_L, 1, _H)),
            _full((1, _H)), _full((1, _H)),
            _full((_H, _H)), _full((1, _H)),
            _full((_H, 2)), _full((1, 2)),
        ],
        out_specs=pl.BlockSpec((1, _C, 2), lambda b: (b, 0, 0)),
        out_shape=jax.ShapeDtypeStruct((_B, _C, 2), jnp.float32),
    )(xs, W1, b1.reshape(1, _H), W2, b2.reshape(1, _H),
      Wg, bg.reshape(_L, 1, _H), gg.reshape(_L, 1, _H), betag.reshape(_L, 1, _H),
      ghln.reshape(1, _H), bhln.reshape(1, _H), Wh1, bh1.reshape(1, _H),
      Wh2, bh2.reshape(1, 2))


def kernel(x, wavelengths, W1, b1, W2, b2, Wg, bg, gg, betag, ghln, bhln,
           Wh1, bh1, Wh2, bh2):
    sort_idx = jnp.argsort(wavelengths)
    xs = x[:, sort_idx, :]
    out_s = _run(xs, W1, b1, W2, b2, Wg, bg, gg, betag, ghln, bhln,
                 Wh1, bh1, Wh2, bh2)
    inv = jnp.argsort(sort_idx)
    out = out_s[:, inv, :]
    return (out[..., 0], out[..., 1])


# speedup vs baseline: 33.9947x; 33.9947x over previous
import functools

import jax
import jax.numpy as jnp
import numpy as np
from jax.experimental import pallas as pl
from jax.experimental.pallas import tpu as pltpu

_B, _C, _FD, _H, _L = 8, 10000, 8, 128, 4
_MIN_LS, _MAX_LS = -7.0, 3.0


def _gelu(v):
    return 0.5 * v * (1.0 + jax.lax.erf(v * jnp.float32(0.7071067811865476)))


def _ln(v, g, b, eps=1e-5):
    mu = jnp.mean(v, axis=-1, keepdims=True)
    var = jnp.mean((v - mu) ** 2, axis=-1, keepdims=True)
    return (v - mu) * jax.lax.rsqrt(var + eps) * g + b


def _main_body(x_ref, W1_ref, b1_ref, W2_ref, b2_ref, Wg_ref, bg_ref, gg_ref,
               betag_ref, ghln_ref, bhln_ref, Wh1_ref, bh1_ref, Wh2_ref, bh2_ref,
               out_ref):
    xb = x_ref[0]
    h = _gelu(jnp.dot(xb, W1_ref[...], preferred_element_type=jnp.float32) + b1_ref[...])
    h = jnp.dot(h, W2_ref[...], preferred_element_type=jnp.float32) + b2_ref[...]

    t = jax.lax.broadcasted_iota(jnp.int32, (_C, 1), 0)
    third = jnp.float32(1.0 / 3.0)
    s6 = jnp.float32(1.0 / np.sqrt(6.0))
    cd = jnp.where((t == 0) | (t == _C - 1), jnp.float32(0.5), third)
    cl = jnp.where(t == 0, jnp.float32(0.0),
                   jnp.where((t == 1) | (t == _C - 1), s6, third))
    cr = jnp.where(t == _C - 1, jnp.float32(0.0),
                   jnp.where((t == 0) | (t == _C - 2), s6, third))

    zrow = jnp.zeros((1, _H), jnp.float32)
    for l in range(_L):
        hl = jnp.dot(h, Wg_ref[l], preferred_element_type=jnp.float32) + bg_ref[l]
        prev = jnp.concatenate([zrow, hl[:-1, :]], axis=0)
        nxt = jnp.concatenate([hl[1:, :], zrow], axis=0)
        agg = cd * hl + cl * prev + cr * nxt
        h = jax.nn.relu(_ln(agg + h, gg_ref[l], betag_ref[l]))

    z = _ln(h, ghln_ref[...], bhln_ref[...])
    z = _gelu(jnp.dot(z, Wh1_ref[...], preferred_element_type=jnp.float32) + bh1_ref[...])
    z2 = jnp.dot(z, Wh2_ref[...], preferred_element_type=jnp.float32) + bh2_ref[...]
    col = jax.lax.broadcasted_iota(jnp.int32, (_C, 2), 1)
    z2 = jnp.where(col == 1, jnp.clip(z2, _MIN_LS, _MAX_LS), z2)
    out_ref[0] = z2


def _full(shape):
    return pl.BlockSpec(shape, lambda b: (0,) * len(shape))


@jax.jit
def _run(xs, W1, b1, W2, b2, Wg, bg, gg, betag, ghln, bhln, Wh1, bh1, Wh2, bh2):
    return pl.pallas_call(
        _main_body,
        grid=(_B,),
        in_specs=[
            pl.BlockSpec((1, _C, _FD), lambda b: (b, 0, 0)),
            _full((_FD, _H)), _full((1, _H)),
            _full((_H, _H)), _full((1, _H)),
            _full((_L, _H, _H)), _full((_L, 1, _H)),
            _full((_L, 1, _H)), _full((_L, 1, _H)),
            _full((1, _H)), _full((1, _H)),
            _full((_H, _H)), _full((1, _H)),
            _full((_H, 2)), _full((1, 2)),
        ],
        out_specs=pl.BlockSpec((1, _C, 2), lambda b: (b, 0, 0)),
        out_shape=jax.ShapeDtypeStruct((_B, _C, 2), jnp.float32),
    )(xs, W1, b1.reshape(1, _H), W2, b2.reshape(1, _H),
      Wg, bg.reshape(_L, 1, _H), gg.reshape(_L, 1, _H), betag.reshape(_L, 1, _H),
      ghln.reshape(1, _H), bhln.reshape(1, _H), Wh1, bh1.reshape(1, _H),
      Wh2, bh2.reshape(1, 2))


def kernel(x, wavelengths, W1, b1, W2, b2, Wg, bg, gg, betag, ghln, bhln,
           Wh1, bh1, Wh2, bh2):
    sort_idx = jnp.argsort(wavelengths)
    xs = x[:, sort_idx, :]
    out_s = _run(xs, W1, b1, W2, b2, Wg, bg, gg, betag, ghln, bhln,
                 Wh1, bh1, Wh2, bh2)
    inv = jnp.argsort(sort_idx)
    out = out_s[:, inv, :]
    return (out[..., 0], out[..., 1])


# speedup vs baseline: 37.2695x; 1.0963x over previous
import functools

import jax
import jax.numpy as jnp
import numpy as np
from jax.experimental import pallas as pl
from jax.experimental.pallas import tpu as pltpu

_B, _C, _FD, _H, _L = 8, 10000, 8, 128, 4
_MIN_LS, _MAX_LS = -7.0, 3.0


def _gelu(v):
    return 0.5 * v * (1.0 + jax.lax.erf(v * jnp.float32(0.7071067811865476)))


def _ln(v, g, b, eps=1e-5):
    mu = jnp.mean(v, axis=-1, keepdims=True)
    var = jnp.mean((v - mu) ** 2, axis=-1, keepdims=True)
    return (v - mu) * jax.lax.rsqrt(var + eps) * g + b


def _main_body(x_ref, W1_ref, b1_ref, W2_ref, b2_ref, Wg_ref, bg_ref, gg_ref,
               betag_ref, ghln_ref, bhln_ref, Wh1_ref, bh1_ref, Wh2_ref, bh2_ref,
               out_ref):
    xb = x_ref[0]
    h = _gelu(jnp.dot(xb, W1_ref[...], preferred_element_type=jnp.float32) + b1_ref[...])
    h = jnp.dot(h, W2_ref[...], preferred_element_type=jnp.float32) + b2_ref[...]

    t = jax.lax.broadcasted_iota(jnp.int32, (_C, 1), 0)
    third = jnp.float32(1.0 / 3.0)
    s6 = jnp.float32(1.0 / np.sqrt(6.0))
    cd = jnp.where((t == 0) | (t == _C - 1), jnp.float32(0.5), third)
    cl = jnp.where(t == 0, jnp.float32(0.0),
                   jnp.where((t == 1) | (t == _C - 1), s6, third))
    cr = jnp.where(t == _C - 1, jnp.float32(0.0),
                   jnp.where((t == 0) | (t == _C - 2), s6, third))

    zrow = jnp.zeros((1, _H), jnp.float32)
    for l in range(_L):
        hl = jnp.dot(h, Wg_ref[l], preferred_element_type=jnp.float32) + bg_ref[l]
        prev = jnp.concatenate([zrow, hl[:-1, :]], axis=0)
        nxt = jnp.concatenate([hl[1:, :], zrow], axis=0)
        agg = cd * hl + cl * prev + cr * nxt
        h = jax.nn.relu(_ln(agg + h, gg_ref[l], betag_ref[l]))

    z = _ln(h, ghln_ref[...], bhln_ref[...])
    z = _gelu(jnp.dot(z, Wh1_ref[...], preferred_element_type=jnp.float32) + bh1_ref[...])
    z2 = jnp.dot(z, Wh2_ref[...], preferred_element_type=jnp.float32) + bh2_ref[...]
    col = jax.lax.broadcasted_iota(jnp.int32, (_C, 2), 1)
    z2 = jnp.where(col == 1, jnp.clip(z2, _MIN_LS, _MAX_LS), z2)
    out_ref[0] = z2


def _full(shape):
    return pl.BlockSpec(shape, lambda b: (0,) * len(shape))


@jax.jit
def _run(xs, W1, b1, W2, b2, Wg, bg, gg, betag, ghln, bhln, Wh1, bh1, Wh2, bh2):
    return pl.pallas_call(
        _main_body,
        grid=(_B,),
        in_specs=[
            pl.BlockSpec((1, _C, _FD), lambda b: (b, 0, 0)),
            _full((_FD, _H)), _full((1, _H)),
            _full((_H, _H)), _full((1, _H)),
            _full((_L, _H, _H)), _full((_L, 1, _H)),
            _full((_L, 1, _H)), _full((_L, 1, _H)),
            _full((1, _H)), _full((1, _H)),
            _full((_H, _H)), _full((1, _H)),
            _full((_H, 2)), _full((1, 2)),
        ],
        out_specs=pl.BlockSpec((1, _C, 2), lambda b: (b, 0, 0)),
        out_shape=jax.ShapeDtypeStruct((_B, _C, 2), jnp.float32),
    )(xs, W1, b1.reshape(1, _H), W2, b2.reshape(1, _H),
      Wg, bg.reshape(_L, 1, _H), gg.reshape(_L, 1, _H), betag.reshape(_L, 1, _H),
      ghln.reshape(1, _H), bhln.reshape(1, _H), Wh1, bh1.reshape(1, _H),
      Wh2, bh2.reshape(1, 2))


def kernel(x, wavelengths, W1, b1, W2, b2, Wg, bg, gg, betag, ghln, bhln,
           Wh1, bh1, Wh2, bh2):
    sort_idx = jnp.argsort(wavelengths)
    xt = jnp.transpose(x, (1, 0, 2)).reshape(_C, _B * _FD)
    xs = jnp.transpose(xt[sort_idx].reshape(_C, _B, _FD), (1, 0, 2))
    out_s = _run(xs, W1, b1, W2, b2, Wg, bg, gg, betag, ghln, bhln,
                 Wh1, bh1, Wh2, bh2)
    inv = jnp.zeros((_C,), jnp.int32).at[sort_idx].set(
        jnp.arange(_C, dtype=jnp.int32))
    out = out_s[:, inv, :]
    return (out[..., 0], out[..., 1])
